# baseline (speedup 1.0000x reference)
import jax
import jax.numpy as jnp
from jax import lax
from jax.experimental import pallas as pl
from jax.experimental.pallas import tpu as pltpu
from jax.experimental.pallas import tpu_sc as plsc

_L = 16


def _pos_body(col_hbm, row_hbm, out_hbm, slab_v, row_v, sem, sem2):
    batches, h = out_hbm.shape[0], out_hbm.shape[1]
    n_xg = out_hbm.shape[2]
    n_cg = out_hbm.shape[3]
    n_cgh = n_cg // 2

    wid = lax.axis_index("s") * 2 + lax.axis_index("c")

    col_copies = []
    for xg in range(n_xg):
        for cgl in range(n_cgh):
            src = col_hbm.at[pl.ds(xg * 8, 8), pl.ds(cgl * 128, 128)]
            col_copies.append(pltpu.async_copy(src, slab_v.at[xg, cgl], sem2))

    pltpu.sync_copy(row_hbm.at[pl.ds(0, h)], row_v)

    def fill(xg, carry):
        for cgh in range(n_cgh):
            for ch in range(128 // _L):
                v = row_v[wid, pl.ds(cgh * 128 + ch * _L, _L)]
                for xi in range(8):
                    slab_v[xg, n_cgh + cgh, xi, pl.ds(ch * _L, _L)] = v
        return carry

    lax.fori_loop(0, n_xg, fill, 0)

    for cp in col_copies:
        cp.wait()

    out_copies = []
    for b in range(batches):
        out_copies.append(pltpu.async_copy(slab_v, out_hbm.at[b, wid], sem))
    for cp in out_copies:
        cp.wait()


@jax.jit
def kernel(pixel_values, row_embeddings, column_embeddings):
    b = pixel_values.shape[0]
    h, w = pixel_values.shape[-2], pixel_values.shape[-1]
    d = column_embeddings.shape[-1]
    n_xg, n_cg = w // 8, (2 * d) // 128

    run = pl.kernel(
        _pos_body,
        out_type=jax.ShapeDtypeStruct((b, h, n_xg, n_cg, 8, 128), jnp.float32),
        mesh=plsc.VectorSubcoreMesh(core_axis_name="c", subcore_axis_name="s"),
        compiler_params=pltpu.CompilerParams(
            use_tc_tiling_on_sc=False, needs_layout_passes=False
        ),
        scratch_types=[
            pltpu.VMEM((n_xg, n_cg, 8, 128), jnp.float32),
            pltpu.VMEM((h, d), jnp.float32),
            pltpu.SemaphoreType.DMA,
            pltpu.SemaphoreType.DMA,
        ],
    )
    out6 = run(column_embeddings, row_embeddings)
    return out6.transpose(0, 3, 5, 1, 2, 4).reshape(b, 2 * d, h, w)


# speedup vs baseline: 1.5496x; 1.5496x over previous
import jax
import jax.numpy as jnp
from jax.experimental import pallas as pl
from jax.experimental.pallas import tpu as pltpu


def _pos_body(col_ref, row_ref, out_ref):
    yb = out_ref.shape[1]
    w = out_ref.shape[2]
    d = col_ref.shape[1]
    col = col_ref[:w, :]
    row = row_ref[:yb, :]
    colb = jnp.broadcast_to(col[None, None, :, :], (1, yb, w, d))
    rowb = jnp.broadcast_to(row[None, :, None, :], (1, yb, w, d))
    out_ref[...] = jnp.concatenate([colb, rowb], axis=-1)


@jax.jit
def kernel(pixel_values, row_embeddings, column_embeddings):
    b = pixel_values.shape[0]
    h, w = pixel_values.shape[-2], pixel_values.shape[-1]
    d = column_embeddings.shape[-1]
    yb = 8

    out = pl.pallas_call(
        _pos_body,
        grid=(b, h // yb),
        in_specs=[
            pl.BlockSpec((w, d), lambda i, j: (0, 0)),
            pl.BlockSpec((yb, d), lambda i, j: (j, 0)),
        ],
        out_specs=pl.BlockSpec((1, yb, w, 2 * d), lambda i, j: (i, j, 0, 0)),
        out_shape=jax.ShapeDtypeStruct((b, h, w, 2 * d), jnp.float32),
        compiler_params=pltpu.CompilerParams(
            dimension_semantics=("parallel", "parallel"),
        ),
    )(column_embeddings, row_embeddings)
    return out.transpose(0, 3, 1, 2)


# speedup vs baseline: 3.9276x; 2.5346x over previous
import jax
import jax.numpy as jnp
from jax.experimental import pallas as pl
from jax.experimental.pallas import tpu as pltpu


def _pos_body(col_ref, row_ref, out_ref):
    yb = out_ref.shape[1]
    w = out_ref.shape[2]
    d = col_ref.shape[1]
    col = col_ref[:w, :]
    row = row_ref[:yb, :]
    colb = jnp.broadcast_to(col[None, None, :, :], (1, yb, w, d))
    rowb = jnp.broadcast_to(row[None, :, None, :], (1, yb, w, d))
    out_ref[...] = jnp.concatenate([colb, rowb], axis=-1)


@jax.jit
def kernel(pixel_values, row_embeddings, column_embeddings):
    b = pixel_values.shape[0]
    h, w = pixel_values.shape[-2], pixel_values.shape[-1]
    d = column_embeddings.shape[-1]
    yb = h

    out = pl.pallas_call(
        _pos_body,
        grid=(b * h // (yb * 1),) if False else (b,),
        in_specs=[
            pl.BlockSpec((w, d), lambda i: (0, 0)),
            pl.BlockSpec((yb, d), lambda i: (0, 0)),
        ],
        out_specs=pl.BlockSpec((1, yb, w, 2 * d), lambda i: (i, 0, 0, 0)),
        out_shape=jax.ShapeDtypeStruct((b, h, w, 2 * d), jnp.float32),
        compiler_params=pltpu.CompilerParams(
            dimension_semantics=("parallel",),
        ),
    )(column_embeddings, row_embeddings)
    return out.transpose(0, 3, 1, 2)


# speedup vs baseline: 4.0463x; 1.0302x over previous
import jax
import jax.numpy as jnp
from jax.experimental import pallas as pl
from jax.experimental.pallas import tpu as pltpu


def _pos_body(col_ref, row_ref, out_ref):
    nb, yb, w = out_ref.shape[:3]
    d = col_ref.shape[1]
    col = col_ref[:w, :]
    row = row_ref[:yb, :]
    colb = jnp.broadcast_to(col[None, None, :, :], (nb, yb, w, d))
    rowb = jnp.broadcast_to(row[None, :, None, :], (nb, yb, w, d))
    out_ref[...] = jnp.concatenate([colb, rowb], axis=-1)


@jax.jit
def kernel(pixel_values, row_embeddings, column_embeddings):
    b = pixel_values.shape[0]
    h, w = pixel_values.shape[-2], pixel_values.shape[-1]
    d = column_embeddings.shape[-1]
    nb = 2

    out = pl.pallas_call(
        _pos_body,
        grid=(b // nb,),
        in_specs=[
            pl.BlockSpec((w, d), lambda i: (0, 0)),
            pl.BlockSpec((h, d), lambda i: (0, 0)),
        ],
        out_specs=pl.BlockSpec((nb, h, w, 2 * d), lambda i: (i, 0, 0, 0)),
        out_shape=jax.ShapeDtypeStruct((b, h, w, 2 * d), jnp.float32),
        compiler_params=pltpu.CompilerParams(
            dimension_semantics=("parallel",),
        ),
    )(column_embeddings, row_embeddings)
    return out.transpose(0, 3, 1, 2)
